# baseline (speedup 1.0000x reference)
import functools

import jax
import jax.numpy as jnp
from jax import lax
from jax.experimental import pallas as pl
from jax.experimental.pallas import tpu as pltpu
from jax.experimental.pallas import tpu_sc as plsc

N_NODES = 10000
E = 160000
R0 = 5.0

PW = 104
NC = 2
NS = 16
CHUNK = 128
GRP = 4
CPT = 80
EPT = CPT * CHUNK
EP = NS * EPT
RPT = 632
NPAD = NS * RPT
TCB = 2048


def _leaky(x):
    return jnp.where(x >= 0, x, 0.1 * x)


def _spread(x, reps):
    k = x.shape[1]
    sel = (lax.broadcasted_iota(jnp.int32, (k, k * reps), 1) // reps ==
           lax.broadcasted_iota(jnp.int32, (k, k * reps), 0)
           ).astype(jnp.float32)
    return _dot(x, sel)


def _tile(x, reps):
    k = x.shape[1]
    sel = (lax.broadcasted_iota(jnp.int32, (k, reps * k), 1) % k ==
           lax.broadcasted_iota(jnp.int32, (k, reps * k), 0)
           ).astype(jnp.float32)
    return _dot(x, sel)


def _dot(a, b):
    return lax.dot(a, b, precision=lax.Precision.HIGHEST,
                   preferred_element_type=jnp.float32)


def _edge_phi_body(r_ref, w0t, b0, wdt, w1t, b1, w2t, b2, w3t, b3, wvt, wddt,
                   out_ref):
    r = r_ref[...]
    d = jnp.sqrt(jnp.sum(r * r, axis=1, keepdims=True) + 1e-12)
    x = d * (1.0 / R0)
    mu = lax.broadcasted_iota(jnp.int32, (1, 8), 1).astype(jnp.float32) / 7.0
    enc = jnp.exp(-0.5 * ((x - mu) * 8.0) ** 2)
    h = _dot(enc, w0t[...]) + b0[...]
    direct = _dot(h, wdt[...])
    y = _leaky(_dot(h, w1t[...]) + b1[...])
    y = _leaky(_dot(y, w2t[...]) + b2[...])
    y = _dot(y, w3t[...]) + b3[...]
    rad = direct + y
    gv = _dot(rad, wvt[...])
    gd = _dot(rad, wddt[...])
    rs = r * (7.0 / R0)
    n2 = jnp.sum(rs * rs, axis=1, keepdims=True)
    rh = rs / jnp.sqrt(1.0 + n2)
    p_v = _spread(gv, 3) * _tile(rh, 16)
    rr9 = _spread(rh, 3) * _tile(rh, 3)
    p_d = _spread(gd, 9) * _tile(rr9, 16)
    out_ref[0] = jnp.concatenate([rad, p_v, p_d[:, :PW - 64]], axis=1)
    out_ref[1] = p_d[:, PW - 64:]


def _edge_payload(r_pad, *ws):
    wspecs = [pl.BlockSpec(w.shape, lambda i: (0, 0)) for w in ws]
    return pl.pallas_call(
        _edge_phi_body,
        grid=(EP // TCB,),
        in_specs=[pl.BlockSpec((TCB, 3), lambda i: (i, 0))] + wspecs,
        out_specs=pl.BlockSpec((2, TCB, PW), lambda i: (0, i, 0)),
        out_shape=jax.ShapeDtypeStruct((2, EP, PW), jnp.float32),
    )(r_pad, *ws)


@functools.cache
def _build_segment_sum_sc():
    mesh = plsc.VectorSubcoreMesh(
        core_axis_name="c", subcore_axis_name="s",
        num_cores=NC, num_subcores=NS)
    return pl.kernel(
        _segment_sum_sc_body,
        out_type=jax.ShapeDtypeStruct((NC, NPAD, PW), jnp.float32),
        mesh=mesh,
        scratch_types=[
            pltpu.VMEM_SHARED((NPAD, PW), jnp.float32),
            pltpu.VMEM((GRP * CHUNK, PW), jnp.float32),
            pltpu.VMEM((CPT, CHUNK), jnp.int32),
            pltpu.SemaphoreType.DMA,
        ],
        compiler_params=pltpu.CompilerParams(use_tc_tiling_on_sc=False),
    )


def _segment_sum_sc_body(pay_hbm, idx_hbm, zeros_hbm, out_hbm, acc, pbuf,
                         idxbuf, sem):
    c = lax.axis_index("c")
    s = lax.axis_index("s")
    pltpu.sync_copy(idx_hbm.at[s], idxbuf)
    pltpu.sync_copy(zeros_hbm, acc.at[pl.ds(s * RPT, RPT)])
    plsc.subcore_barrier()

    def body(g, carry):
        base = s * EPT + g * (GRP * CHUNK)
        pltpu.sync_copy(pay_hbm.at[c, pl.ds(base, GRP * CHUNK)], pbuf)
        descs = [
            pltpu.async_copy(
                pbuf.at[pl.ds(k * CHUNK, CHUNK)],
                acc.at[idxbuf.at[g * GRP + k]],
                sem, add=True)
            for k in range(GRP)
        ]
        for dsc in descs:
            dsc.wait()
        return carry

    lax.fori_loop(0, CPT // GRP, body, 0)
    plsc.subcore_barrier()
    pltpu.sync_copy(acc.at[pl.ds(s * RPT, RPT)],
                    out_hbm.at[c, pl.ds(s * RPT, RPT)])


def kernel(r_ij, edge_index, W0, b0, Wd, W1, b1, W2, b2, W3, b3, Wv, Wdd):
    src = edge_index[0].astype(jnp.int32)
    r_pad = jnp.zeros((EP, 3), jnp.float32).at[:E].set(r_ij)
    idx = (jnp.full((EP,), N_NODES, jnp.int32).at[:E].set(src)
           .reshape(NS, CPT, CHUNK))
    ws = (W0.T, b0.reshape(1, -1), Wd.T, W1.T, b1.reshape(1, -1), W2.T,
          b2.reshape(1, -1), W3.T, b3.reshape(1, -1), Wv.T, Wdd.T)
    payload = _edge_payload(r_pad, *ws)
    acc = _build_segment_sum_sc()(payload, idx,
                                  jnp.zeros((RPT, PW), jnp.float32))
    a_a = acc[0, :N_NODES, 0:16]
    out_v = acc[0, :N_NODES, 16:64].reshape(N_NODES, 16, 3)
    out_d = (jnp.concatenate([acc[0, :N_NODES, 64:PW], acc[1, :N_NODES]],
                             axis=1).reshape(N_NODES, 16, 3, 3))
    return (a_a, out_v, out_d)


# speedup vs baseline: 3.8616x; 3.8616x over previous
import functools

import jax
import jax.numpy as jnp
from jax import lax
from jax.experimental import pallas as pl
from jax.experimental.pallas import tpu as pltpu
from jax.experimental.pallas import tpu_sc as plsc

N_NODES = 10000
E = 160000
R0 = 5.0

PW = 80
NC = 2
NS = 16
CHUNK = 128
GRP = 4
CPT = 80
EPT = CPT * CHUNK
EP = NS * EPT
RPT = 632
NPAD = NS * RPT
TCB = 8192


def _leaky(x):
    return jnp.where(x >= 0, x, 0.1 * x)


def _dot(a, b):
    return lax.dot(a, b, precision=lax.Precision.HIGHEST,
                   preferred_element_type=jnp.float32)


def _edge_phi_body(r_ref, w0, b0, wd, w1, b1, w2, b2, w3, b3, wv, wdd,
                   out_ref):
    rt = r_ref[...]
    d2 = jnp.sum(rt * rt, axis=0, keepdims=True)
    x = jnp.sqrt(d2 + 1e-12) * (1.0 / R0)
    mu = lax.broadcasted_iota(jnp.int32, (8, 1), 0).astype(jnp.float32) / 7.0
    enc = jnp.exp(-0.5 * ((x - mu) * 8.0) ** 2)
    h = _dot(w0[...], enc) + b0[...]
    direct = _dot(wd[...], h)
    y = _leaky(_dot(w1[...], h) + b1[...])
    y = _leaky(_dot(w2[...], y) + b2[...])
    y = _dot(w3[...], y) + b3[...]
    rad = direct + y
    gv = _dot(wv[...], rad)
    gd = _dot(wdd[...], rad)
    rs = rt * (7.0 / R0)
    n2 = jnp.sum(rs * rs, axis=0, keepdims=True)
    rh = rs / jnp.sqrt(1.0 + n2)
    r0_, r1_, r2_ = rh[0:1], rh[1:2], rh[2:3]
    p0 = jnp.concatenate(
        [rad, gv * r0_, gv * r1_, gv * r2_, gd * (r0_ * r0_)], axis=0)
    p1 = jnp.concatenate(
        [gd * (r0_ * r1_), gd * (r0_ * r2_), gd * (r1_ * r1_),
         gd * (r1_ * r2_), gd * (r2_ * r2_)], axis=0)
    out_ref[0] = p0.T
    out_ref[1] = p1.T


def _edge_payload(r_t, *ws):
    wspecs = [pl.BlockSpec(w.shape, lambda i: (0, 0)) for w in ws]
    return pl.pallas_call(
        _edge_phi_body,
        grid=(EP // TCB,),
        in_specs=[pl.BlockSpec((3, TCB), lambda i: (0, i))] + wspecs,
        out_specs=pl.BlockSpec((2, TCB, PW), lambda i: (0, i, 0)),
        out_shape=jax.ShapeDtypeStruct((2, EP, PW), jnp.float32),
    )(r_t, *ws)


@functools.cache
def _build_segment_sum_sc():
    mesh = plsc.VectorSubcoreMesh(
        core_axis_name="c", subcore_axis_name="s",
        num_cores=NC, num_subcores=NS)
    return pl.kernel(
        _segment_sum_sc_body,
        out_type=jax.ShapeDtypeStruct((NC, NPAD, PW), jnp.float32),
        mesh=mesh,
        scratch_types=[
            pltpu.VMEM_SHARED((NPAD, PW), jnp.float32),
            pltpu.VMEM((GRP * CHUNK, PW), jnp.float32),
            pltpu.VMEM((CPT, CHUNK), jnp.int32),
            pltpu.SemaphoreType.DMA,
        ],
        compiler_params=pltpu.CompilerParams(use_tc_tiling_on_sc=False),
    )


def _segment_sum_sc_body(pay_hbm, idx_hbm, zeros_hbm, out_hbm, acc, pbuf,
                         idxbuf, sem):
    c = lax.axis_index("c")
    s = lax.axis_index("s")
    pltpu.sync_copy(idx_hbm.at[s], idxbuf)
    pltpu.sync_copy(zeros_hbm, acc.at[pl.ds(s * RPT, RPT)])
    plsc.subcore_barrier()

    def body(g, carry):
        base = s * EPT + g * (GRP * CHUNK)
        pltpu.sync_copy(pay_hbm.at[c, pl.ds(base, GRP * CHUNK)], pbuf)
        descs = [
            pltpu.async_copy(
                pbuf.at[pl.ds(k * CHUNK, CHUNK)],
                acc.at[idxbuf.at[g * GRP + k]],
                sem, add=True)
            for k in range(GRP)
        ]
        for dsc in descs:
            dsc.wait()
        return carry

    lax.fori_loop(0, CPT // GRP, body, 0)
    plsc.subcore_barrier()
    pltpu.sync_copy(acc.at[pl.ds(s * RPT, RPT)],
                    out_hbm.at[c, pl.ds(s * RPT, RPT)])


def kernel(r_ij, edge_index, W0, b0, Wd, W1, b1, W2, b2, W3, b3, Wv, Wdd):
    src = edge_index[0].astype(jnp.int32)
    r_t = jnp.zeros((3, EP), jnp.float32).at[:, :E].set(r_ij.T)
    idx = (jnp.full((EP,), N_NODES, jnp.int32).at[:E].set(src)
           .reshape(NS, CPT, CHUNK))
    ws = (W0, b0.reshape(-1, 1), Wd, W1, b1.reshape(-1, 1), W2,
          b2.reshape(-1, 1), W3, b3.reshape(-1, 1), Wv, Wdd)
    payload = _edge_payload(r_t, *ws)
    acc = _build_segment_sum_sc()(payload, idx,
                                  jnp.zeros((RPT, PW), jnp.float32))
    o0 = acc[0, :N_NODES]
    o1 = acc[1, :N_NODES]
    a_a = o0[:, 0:16]
    out_v = o0[:, 16:64].reshape(N_NODES, 3, 16).transpose(0, 2, 1)
    d6 = (jnp.concatenate([o0[:, 64:80], o1], axis=1)
          .reshape(N_NODES, 6, 16).transpose(0, 2, 1))
    sym = jnp.array([[0, 1, 2], [1, 3, 4], [2, 4, 5]], jnp.int32)
    out_d = d6[:, :, sym]
    return (a_a, out_v, out_d)
